# Initial kernel scaffold; baseline (speedup 1.0000x reference)
import jax
import jax.numpy as jnp
from jax.experimental import pallas as pl


def kernel(x_user, x_recipe, user_node_id, recipe_node_id, edge_index_u2r, edge_index_r2u, W_user, b_user, W_recipe, b_recipe, emb_user, emb_recipe, Wl1_u2r, Wr1_u2r, Wl1_r2u, Wr1_r2u, Wl2_u2r, Wr2_u2r, Wl2_r2u, Wr2_r2u, bl1_u2r, bl1_r2u, bl2_u2r, bl2_r2u):
    raise NotImplementedError("write your pallas kernel here")


# speedup vs baseline: 4.0122x; 4.0122x over previous
import functools

import jax
import jax.numpy as jnp
from jax import lax
from jax.experimental import pallas as pl
from jax.experimental.pallas import tpu as pltpu
from jax.experimental.pallas import tpu_sc as plsc

_N = 10000
_E = 320000
_H = 128
_NT = 16
_CK = 80
_NCH = _E // _NT // _CK
_CW = 16
_ZST = 640
_OST = 624
_ACC = _NT * _ZST


def _fill_f32(ref, nrows, ncols, val):
    v = jnp.full((16,), val, jnp.float32)

    def row(r, c):
        for cb in range(ncols // 16):
            ref[r, pl.ds(cb * 16, 16)] = v
        return c

    lax.fori_loop(0, nrows, row, 0)


_HH = _H // 2


def _make_agg(with_counts):
    mesh = plsc.VectorSubcoreMesh(
        core_axis_name="c", subcore_axis_name="s", num_cores=2,
        num_subcores=_NT)

    out_type = [jax.ShapeDtypeStruct((2, _N, _HH), jnp.float32),
                jax.ShapeDtypeStruct((2, _N, _HH), jnp.float32)]
    if with_counts:
        out_type.append(jax.ShapeDtypeStruct((2, _N, _CW), jnp.float32))

    scratch = [
        pltpu.VMEM((_NCH, _CK), jnp.int32),
        pltpu.VMEM((_NCH, _CK), jnp.int32),
        pltpu.VMEM((_CK, _HH), jnp.float32),
        pltpu.VMEM((_CK, _CW), jnp.float32),
        pltpu.VMEM_SHARED((_ACC, _HH), jnp.float32),
        pltpu.VMEM_SHARED((_ACC, _CW), jnp.float32),
        pltpu.SemaphoreType.DMA,
    ]

    def body(tab_u0, tab_u1, tab_r0, tab_r1, src_all, dst_all, *rest):
        if with_counts:
            (out0, out1, out_cnt, src_v, dst_v, rows_v, ones_v, acc, cacc,
             sem) = rest
        else:
            out0, out1, src_v, dst_v, rows_v, ones_v, acc, cacc, sem = rest
            out_cnt = None
        c = lax.axis_index("c")
        s = lax.axis_index("s")

        pltpu.sync_copy(src_all.at[c, s], src_v)
        pltpu.sync_copy(dst_all.at[c, s], dst_v)

        def stripe_out(src_ref, dst_ref):
            obase = s * _OST
            pltpu.sync_copy(src_ref.at[pl.ds(obase, _OST)],
                            dst_ref.at[c, pl.ds(obase, _OST)])

            @pl.when(s == _NT - 1)
            def _():
                tbase = _NT * _OST
                pltpu.sync_copy(src_ref.at[pl.ds(tbase, _N - _NT * _OST)],
                                dst_ref.at[c, pl.ds(tbase, _N - _NT * _OST)])

        def half_pass(tab, out, count_pass):
            _fill_f32(rows_v, _CK, _HH, 0.0)
            if count_pass:
                _fill_f32(ones_v, _CK, _CW, 0.0)
            zbase = s * _ZST
            for b in range(_ZST // _CK):
                pltpu.sync_copy(rows_v, acc.at[pl.ds(zbase + b * _CK, _CK)])
                if count_pass:
                    pltpu.sync_copy(ones_v,
                                    cacc.at[pl.ds(zbase + b * _CK, _CK)])
            if count_pass:
                _fill_f32(ones_v, _CK, _CW, 1.0)
            plsc.subcore_barrier()

            def chunk(j, carry):
                pltpu.async_copy(tab.at[src_v.at[j]], rows_v, sem).wait()
                pltpu.sync_copy(rows_v, acc.at[dst_v.at[j]], add=True)
                if count_pass:
                    pltpu.sync_copy(ones_v, cacc.at[dst_v.at[j]], add=True)
                return carry

            lax.fori_loop(0, _NCH, chunk, 0)
            plsc.subcore_barrier()

            stripe_out(acc, out)
            if count_pass:
                stripe_out(cacc, out_cnt)
            plsc.subcore_barrier()

        def both_halves(tab0, tab1):
            half_pass(tab0, out0, with_counts)
            half_pass(tab1, out1, False)

        @pl.when(c == 0)
        def _():
            both_halves(tab_u0, tab_u1)

        @pl.when(c == 1)
        def _():
            both_halves(tab_r0, tab_r1)

    return pl.kernel(body, out_type=out_type, mesh=mesh,
                     scratch_types=scratch,
                     compiler_params=pltpu.CompilerParams(
                         use_tc_tiling_on_sc=False))


_agg_with_counts = _make_agg(True)
_agg_no_counts = _make_agg(False)


_BR = 1000


def _encode_body(xu, xr, Wu, Wr, bu, br, eu, er, ou, outr):
    hp = jax.lax.Precision.HIGHEST
    ou[...] = (jnp.dot(xu[...], Wu[...], precision=hp,
                       preferred_element_type=jnp.float32)
               + bu[...] + eu[...])
    outr[...] = (jnp.dot(xr[...], Wr[...], precision=hp,
                         preferred_element_type=jnp.float32)
                 + br[...] + er[...])


def _encode(xu, xr, Wu, Wr, bu, br, eu, er):
    grid = (_N // _BR,)
    fu = xu.shape[1]
    fr = xr.shape[1]
    return pl.pallas_call(
        _encode_body,
        grid=grid,
        in_specs=[
            pl.BlockSpec((_BR, fu), lambda i: (i, 0)),
            pl.BlockSpec((_BR, fr), lambda i: (i, 0)),
            pl.BlockSpec((fu, _H), lambda i: (0, 0)),
            pl.BlockSpec((fr, _H), lambda i: (0, 0)),
            pl.BlockSpec((1, _H), lambda i: (0, 0)),
            pl.BlockSpec((1, _H), lambda i: (0, 0)),
            pl.BlockSpec((_BR, _H), lambda i: (i, 0)),
            pl.BlockSpec((_BR, _H), lambda i: (i, 0)),
        ],
        out_specs=[
            pl.BlockSpec((_BR, _H), lambda i: (i, 0)),
            pl.BlockSpec((_BR, _H), lambda i: (i, 0)),
        ],
        out_shape=[
            jax.ShapeDtypeStruct((_N, _H), jnp.float32),
            jax.ShapeDtypeStruct((_N, _H), jnp.float32),
        ],
    )(xu, xr, Wu, Wr, bu, br, eu, er)


def _make_combine(apply_relu):
    def body(sums0, sums1, cnts, pu, pr, Wl_u, Wr_u, Wl_r, Wr_r, bl_u, bl_r,
             ou, outr):
        hp = jax.lax.Precision.HIGHEST
        sum_u = jnp.concatenate([sums0[0], sums1[0]], axis=-1)
        sum_r = jnp.concatenate([sums0[1], sums1[1]], axis=-1)
        mean_u = sum_u / jnp.maximum(cnts[0, :, 0:1], 1.0)
        mean_r = sum_r / jnp.maximum(cnts[1, :, 0:1], 1.0)
        hu = (jnp.dot(mean_u, Wl_u[...], precision=hp,
                      preferred_element_type=jnp.float32) + bl_u[...]
              + jnp.dot(pu[...], Wr_u[...], precision=hp,
                        preferred_element_type=jnp.float32))
        hr = (jnp.dot(mean_r, Wl_r[...], precision=hp,
                      preferred_element_type=jnp.float32) + bl_r[...]
              + jnp.dot(pr[...], Wr_r[...], precision=hp,
                        preferred_element_type=jnp.float32))
        if apply_relu:
            hu = jnp.maximum(hu, 0.0)
            hr = jnp.maximum(hr, 0.0)
        ou[...] = hu
        outr[...] = hr

    def run(sums0, sums1, cnts, pu, pr, Wl_u, Wr_u, Wl_r, Wr_r, bl_u, bl_r):
        grid = (_N // _BR,)
        return pl.pallas_call(
            body,
            grid=grid,
            in_specs=[
                pl.BlockSpec((2, _BR, _HH), lambda i: (0, i, 0)),
                pl.BlockSpec((2, _BR, _HH), lambda i: (0, i, 0)),
                pl.BlockSpec((2, _BR, _CW), lambda i: (0, i, 0)),
                pl.BlockSpec((_BR, _H), lambda i: (i, 0)),
                pl.BlockSpec((_BR, _H), lambda i: (i, 0)),
                pl.BlockSpec((_H, _H), lambda i: (0, 0)),
                pl.BlockSpec((_H, _H), lambda i: (0, 0)),
                pl.BlockSpec((_H, _H), lambda i: (0, 0)),
                pl.BlockSpec((_H, _H), lambda i: (0, 0)),
                pl.BlockSpec((1, _H), lambda i: (0, 0)),
                pl.BlockSpec((1, _H), lambda i: (0, 0)),
            ],
            out_specs=[
                pl.BlockSpec((_BR, _H), lambda i: (i, 0)),
                pl.BlockSpec((_BR, _H), lambda i: (i, 0)),
            ],
            out_shape=[
                jax.ShapeDtypeStruct((_N, _H), jnp.float32),
                jax.ShapeDtypeStruct((_N, _H), jnp.float32),
            ],
        )(sums0, sums1, cnts, pu, pr, Wl_u, Wr_u, Wl_r, Wr_r, bl_u, bl_r)

    return run


_combine_relu = _make_combine(True)
_combine_lin = _make_combine(False)


def kernel(x_user, x_recipe, user_node_id, recipe_node_id,
           edge_index_u2r, edge_index_r2u,
           W_user, b_user, W_recipe, b_recipe, emb_user, emb_recipe,
           Wl1_u2r, Wr1_u2r, Wl1_r2u, Wr1_r2u,
           Wl2_u2r, Wr2_u2r, Wl2_r2u, Wr2_r2u,
           bl1_u2r, bl1_r2u, bl2_u2r, bl2_r2u):
    fu = x_user.shape[1]
    fu_pad = (-fu) % 8
    xu = jnp.pad(x_user, ((0, 0), (0, fu_pad)))
    Wu = jnp.pad(W_user, ((0, fu_pad), (0, 0)))
    eu = jnp.take(emb_user, user_node_id, axis=0)
    er = jnp.take(emb_recipe, recipe_node_id, axis=0)

    h_u, h_r = _encode(xu, x_recipe, Wu, W_recipe,
                       b_user.reshape(1, _H), b_recipe.reshape(1, _H),
                       eu, er)

    src_all = jnp.stack([edge_index_r2u[0], edge_index_u2r[0]]
                        ).astype(jnp.int32).reshape(2, _NT, _NCH, _CK)
    dst_all = jnp.stack([edge_index_r2u[1], edge_index_u2r[1]]
                        ).astype(jnp.int32).reshape(2, _NT, _NCH, _CK)

    s1a, s1b, cnts = _agg_with_counts(h_r[:, :_HH], h_r[:, _HH:],
                                      h_u[:, :_HH], h_u[:, _HH:],
                                      src_all, dst_all)
    h_u1, h_r1 = _combine_relu(s1a, s1b, cnts, h_u, h_r,
                               Wl1_r2u, Wr1_r2u, Wl1_u2r, Wr1_u2r,
                               bl1_r2u.reshape(1, _H),
                               bl1_u2r.reshape(1, _H))

    s2a, s2b = _agg_no_counts(h_r1[:, :_HH], h_r1[:, _HH:],
                              h_u1[:, :_HH], h_u1[:, _HH:],
                              src_all, dst_all)
    h_u2, h_r2 = _combine_lin(s2a, s2b, cnts, h_u1, h_r1,
                              Wl2_r2u, Wr2_r2u, Wl2_u2r, Wr2_u2r,
                              bl2_r2u.reshape(1, _H),
                              bl2_u2r.reshape(1, _H))
    return (h_u2, h_r2)


# speedup vs baseline: 7.2815x; 1.8149x over previous
import functools

import jax
import jax.numpy as jnp
from jax import lax
from jax.experimental import pallas as pl
from jax.experimental.pallas import tpu as pltpu
from jax.experimental.pallas import tpu_sc as plsc

_N = 10000
_E = 320000
_H = 128
_NT = 16
_CK = 125
_NCH = _E // _NT // _CK
_CW = 16
_ZST = 625
_OST = 624
_ACC = _NT * _ZST


def _fill_f32(ref, nrows, ncols, val):
    v = jnp.full((16,), val, jnp.float32)

    def row(r, c):
        for cb in range(ncols // 16):
            ref[r, pl.ds(cb * 16, 16)] = v
        return c

    lax.fori_loop(0, nrows, row, 0)


_HH = _H // 2


def _make_agg(with_counts):
    mesh = plsc.VectorSubcoreMesh(
        core_axis_name="c", subcore_axis_name="s", num_cores=2,
        num_subcores=_NT)

    out_type = [jax.ShapeDtypeStruct((2, _N, _HH), jnp.float32),
                jax.ShapeDtypeStruct((2, _N, _HH), jnp.float32)]
    if with_counts:
        out_type.append(jax.ShapeDtypeStruct((2, _N, _CW), jnp.float32))

    scratch = [
        pltpu.VMEM((_NCH, _CK), jnp.int32),
        pltpu.VMEM((_NCH, _CK), jnp.int32),
        pltpu.VMEM((_CK, _HH), jnp.float32),
        pltpu.VMEM((_CK, _HH), jnp.float32),
        pltpu.VMEM((_CK, _HH), jnp.float32),
        pltpu.VMEM((_CK, _CW), jnp.float32),
        pltpu.VMEM_SHARED((_ACC, _HH), jnp.float32),
        pltpu.VMEM_SHARED((_ACC, _CW), jnp.float32),
        pltpu.SemaphoreType.DMA,
        pltpu.SemaphoreType.DMA,
    ]

    def body(tab_u0, tab_u1, tab_r0, tab_r1, src_all, dst_all, *rest):
        if with_counts:
            (out0, out1, out_cnt, src_v, dst_v, rows0, rows1, zbuf, ones_v,
             acc, cacc, sem0, sem1) = rest
        else:
            (out0, out1, src_v, dst_v, rows0, rows1, zbuf, ones_v, acc, cacc,
             sem0, sem1) = rest
            out_cnt = None
        c = lax.axis_index("c")
        s = lax.axis_index("s")
        _fill_f32(zbuf, _CK, _HH, 0.0)

        pltpu.sync_copy(src_all.at[c, s], src_v)
        pltpu.sync_copy(dst_all.at[c, s], dst_v)

        def stripe_out(src_ref, dst_ref):
            obase = s * _OST
            pltpu.sync_copy(src_ref.at[pl.ds(obase, _OST)],
                            dst_ref.at[c, pl.ds(obase, _OST)])

            @pl.when(s == _NT - 1)
            def _():
                tbase = _NT * _OST
                pltpu.sync_copy(src_ref.at[pl.ds(tbase, _N - _NT * _OST)],
                                dst_ref.at[c, pl.ds(tbase, _N - _NT * _OST)])

        def half_pass(tab, out, count_pass):
            if count_pass:
                _fill_f32(ones_v, _CK, _CW, 0.0)
            zbase = s * _ZST
            for b in range(_ZST // _CK):
                pltpu.sync_copy(zbuf, acc.at[pl.ds(zbase + b * _CK, _CK)])
                if count_pass:
                    pltpu.sync_copy(ones_v,
                                    cacc.at[pl.ds(zbase + b * _CK, _CK)])
            if count_pass:
                _fill_f32(ones_v, _CK, _CW, 1.0)
            plsc.subcore_barrier()

            pltpu.async_copy(tab.at[src_v.at[0]], rows0, sem0)

            def pair(jj, carry):
                j0 = 2 * jj
                pltpu.async_copy(tab.at[src_v.at[j0 + 1]], rows1, sem1)
                pltpu.make_async_copy(tab.at[src_v.at[j0]], rows0,
                                      sem0).wait()
                pltpu.sync_copy(rows0, acc.at[dst_v.at[j0]], add=True)
                if count_pass:
                    pltpu.sync_copy(ones_v, cacc.at[dst_v.at[j0]], add=True)
                jn = jnp.minimum(j0 + 2, _NCH - 1)
                pltpu.async_copy(tab.at[src_v.at[jn]], rows0, sem0)
                pltpu.make_async_copy(tab.at[src_v.at[j0 + 1]], rows1,
                                      sem1).wait()
                pltpu.sync_copy(rows1, acc.at[dst_v.at[j0 + 1]], add=True)
                if count_pass:
                    pltpu.sync_copy(ones_v, cacc.at[dst_v.at[j0 + 1]],
                                    add=True)
                return carry

            lax.fori_loop(0, _NCH // 2, pair, 0)
            pltpu.make_async_copy(tab.at[src_v.at[_NCH - 1]], rows0,
                                  sem0).wait()
            plsc.subcore_barrier()

            stripe_out(acc, out)
            if count_pass:
                stripe_out(cacc, out_cnt)
            plsc.subcore_barrier()

        def both_halves(tab0, tab1):
            half_pass(tab0, out0, with_counts)
            half_pass(tab1, out1, False)

        @pl.when(c == 0)
        def _():
            both_halves(tab_u0, tab_u1)

        @pl.when(c == 1)
        def _():
            both_halves(tab_r0, tab_r1)

    return pl.kernel(body, out_type=out_type, mesh=mesh,
                     scratch_types=scratch,
                     compiler_params=pltpu.CompilerParams(
                         use_tc_tiling_on_sc=False))


_agg_with_counts = _make_agg(True)
_agg_no_counts = _make_agg(False)


_BR = 1000


def _encode_body(xu, xr, Wu, Wr, bu, br, eu, er, ou, outr):
    hp = jax.lax.Precision.HIGHEST
    ou[...] = (jnp.dot(xu[...], Wu[...], precision=hp,
                       preferred_element_type=jnp.float32)
               + bu[...] + eu[...])
    outr[...] = (jnp.dot(xr[...], Wr[...], precision=hp,
                         preferred_element_type=jnp.float32)
                 + br[...] + er[...])


def _encode(xu, xr, Wu, Wr, bu, br, eu, er):
    grid = (_N // _BR,)
    fu = xu.shape[1]
    fr = xr.shape[1]
    return pl.pallas_call(
        _encode_body,
        grid=grid,
        in_specs=[
            pl.BlockSpec((_BR, fu), lambda i: (i, 0)),
            pl.BlockSpec((_BR, fr), lambda i: (i, 0)),
            pl.BlockSpec((fu, _H), lambda i: (0, 0)),
            pl.BlockSpec((fr, _H), lambda i: (0, 0)),
            pl.BlockSpec((1, _H), lambda i: (0, 0)),
            pl.BlockSpec((1, _H), lambda i: (0, 0)),
            pl.BlockSpec((_BR, _H), lambda i: (i, 0)),
            pl.BlockSpec((_BR, _H), lambda i: (i, 0)),
        ],
        out_specs=[
            pl.BlockSpec((_BR, _H), lambda i: (i, 0)),
            pl.BlockSpec((_BR, _H), lambda i: (i, 0)),
        ],
        out_shape=[
            jax.ShapeDtypeStruct((_N, _H), jnp.float32),
            jax.ShapeDtypeStruct((_N, _H), jnp.float32),
        ],
    )(xu, xr, Wu, Wr, bu, br, eu, er)


def _make_combine(apply_relu):
    def body(sums0, sums1, cnts, pu, pr, Wl_u, Wr_u, Wl_r, Wr_r, bl_u, bl_r,
             ou, outr):
        hp = jax.lax.Precision.HIGHEST
        sum_u = jnp.concatenate([sums0[0], sums1[0]], axis=-1)
        sum_r = jnp.concatenate([sums0[1], sums1[1]], axis=-1)
        mean_u = sum_u / jnp.maximum(cnts[0, :, 0:1], 1.0)
        mean_r = sum_r / jnp.maximum(cnts[1, :, 0:1], 1.0)
        hu = (jnp.dot(mean_u, Wl_u[...], precision=hp,
                      preferred_element_type=jnp.float32) + bl_u[...]
              + jnp.dot(pu[...], Wr_u[...], precision=hp,
                        preferred_element_type=jnp.float32))
        hr = (jnp.dot(mean_r, Wl_r[...], precision=hp,
                      preferred_element_type=jnp.float32) + bl_r[...]
              + jnp.dot(pr[...], Wr_r[...], precision=hp,
                        preferred_element_type=jnp.float32))
        if apply_relu:
            hu = jnp.maximum(hu, 0.0)
            hr = jnp.maximum(hr, 0.0)
        ou[...] = hu
        outr[...] = hr

    def run(sums0, sums1, cnts, pu, pr, Wl_u, Wr_u, Wl_r, Wr_r, bl_u, bl_r):
        grid = (_N // _BR,)
        return pl.pallas_call(
            body,
            grid=grid,
            in_specs=[
                pl.BlockSpec((2, _BR, _HH), lambda i: (0, i, 0)),
                pl.BlockSpec((2, _BR, _HH), lambda i: (0, i, 0)),
                pl.BlockSpec((2, _BR, _CW), lambda i: (0, i, 0)),
                pl.BlockSpec((_BR, _H), lambda i: (i, 0)),
                pl.BlockSpec((_BR, _H), lambda i: (i, 0)),
                pl.BlockSpec((_H, _H), lambda i: (0, 0)),
                pl.BlockSpec((_H, _H), lambda i: (0, 0)),
                pl.BlockSpec((_H, _H), lambda i: (0, 0)),
                pl.BlockSpec((_H, _H), lambda i: (0, 0)),
                pl.BlockSpec((1, _H), lambda i: (0, 0)),
                pl.BlockSpec((1, _H), lambda i: (0, 0)),
            ],
            out_specs=[
                pl.BlockSpec((_BR, _H), lambda i: (i, 0)),
                pl.BlockSpec((_BR, _H), lambda i: (i, 0)),
            ],
            out_shape=[
                jax.ShapeDtypeStruct((_N, _H), jnp.float32),
                jax.ShapeDtypeStruct((_N, _H), jnp.float32),
            ],
        )(sums0, sums1, cnts, pu, pr, Wl_u, Wr_u, Wl_r, Wr_r, bl_u, bl_r)

    return run


_combine_relu = _make_combine(True)
_combine_lin = _make_combine(False)


def kernel(x_user, x_recipe, user_node_id, recipe_node_id,
           edge_index_u2r, edge_index_r2u,
           W_user, b_user, W_recipe, b_recipe, emb_user, emb_recipe,
           Wl1_u2r, Wr1_u2r, Wl1_r2u, Wr1_r2u,
           Wl2_u2r, Wr2_u2r, Wl2_r2u, Wr2_r2u,
           bl1_u2r, bl1_r2u, bl2_u2r, bl2_r2u):
    fu = x_user.shape[1]
    fu_pad = (-fu) % 8
    xu = jnp.pad(x_user, ((0, 0), (0, fu_pad)))
    Wu = jnp.pad(W_user, ((0, fu_pad), (0, 0)))
    eu = jnp.take(emb_user, user_node_id, axis=0)
    er = jnp.take(emb_recipe, recipe_node_id, axis=0)

    h_u, h_r = _encode(xu, x_recipe, Wu, W_recipe,
                       b_user.reshape(1, _H), b_recipe.reshape(1, _H),
                       eu, er)

    src_all = jnp.stack([edge_index_r2u[0], edge_index_u2r[0]]
                        ).astype(jnp.int32).reshape(2, _NT, _NCH, _CK)
    dst_all = jnp.stack([edge_index_r2u[1], edge_index_u2r[1]]
                        ).astype(jnp.int32).reshape(2, _NT, _NCH, _CK)

    s1a, s1b, cnts = _agg_with_counts(h_r[:, :_HH], h_r[:, _HH:],
                                      h_u[:, :_HH], h_u[:, _HH:],
                                      src_all, dst_all)
    h_u1, h_r1 = _combine_relu(s1a, s1b, cnts, h_u, h_r,
                               Wl1_r2u, Wr1_r2u, Wl1_u2r, Wr1_u2r,
                               bl1_r2u.reshape(1, _H),
                               bl1_u2r.reshape(1, _H))

    s2a, s2b = _agg_no_counts(h_r1[:, :_HH], h_r1[:, _HH:],
                              h_u1[:, :_HH], h_u1[:, _HH:],
                              src_all, dst_all)
    h_u2, h_r2 = _combine_lin(s2a, s2b, cnts, h_u1, h_r1,
                              Wl2_r2u, Wr2_r2u, Wl2_u2r, Wr2_u2r,
                              bl2_r2u.reshape(1, _H),
                              bl2_u2r.reshape(1, _H))
    return (h_u2, h_r2)


# speedup vs baseline: 7.4125x; 1.0180x over previous
import functools

import jax
import jax.numpy as jnp
from jax import lax
from jax.experimental import pallas as pl
from jax.experimental.pallas import tpu as pltpu
from jax.experimental.pallas import tpu_sc as plsc

_N = 10000
_E = 320000
_H = 128
_NT = 16
_CK = 125
_NCH = _E // _NT // _CK
_CW = 16
_ZST = 625
_OST = 624
_ACC = _NT * _ZST


def _fill_f32(ref, nrows, ncols, val):
    v = jnp.full((16,), val, jnp.float32)

    def row(r, c):
        for cb in range(ncols // 16):
            ref[r, pl.ds(cb * 16, 16)] = v
        return c

    lax.fori_loop(0, nrows, row, 0)


_HH = _H // 2


def _make_agg(with_counts):
    mesh = plsc.VectorSubcoreMesh(
        core_axis_name="c", subcore_axis_name="s", num_cores=2,
        num_subcores=_NT)

    out_type = [jax.ShapeDtypeStruct((2, _N, _HH), jnp.float32),
                jax.ShapeDtypeStruct((2, _N, _HH), jnp.float32)]
    if with_counts:
        out_type.append(jax.ShapeDtypeStruct((2, _N, _CW), jnp.float32))

    scratch = [
        pltpu.VMEM((_NCH, _CK), jnp.int32),
        pltpu.VMEM((_NCH, _CK), jnp.int32),
        [pltpu.VMEM((_CK, _HH), jnp.float32) for _ in range(4)],
        pltpu.VMEM((_CK, _HH), jnp.float32),
        pltpu.VMEM((_CK, _CW), jnp.float32),
        pltpu.VMEM_SHARED((_ACC, _HH), jnp.float32),
        pltpu.VMEM_SHARED((_ACC, _CW), jnp.float32),
        [pltpu.SemaphoreType.DMA for _ in range(4)],
        [pltpu.SemaphoreType.DMA for _ in range(4)],
    ]

    def body(tab_u0, tab_u1, tab_r0, tab_r1, src_all, dst_all, *rest):
        if with_counts:
            (out0, out1, out_cnt, src_v, dst_v, rows, zbuf, ones_v,
             acc, cacc, gsem, ssem) = rest
        else:
            (out0, out1, src_v, dst_v, rows, zbuf, ones_v, acc, cacc,
             gsem, ssem) = rest
            out_cnt = None
        c = lax.axis_index("c")
        s = lax.axis_index("s")
        _fill_f32(zbuf, _CK, _HH, 0.0)

        pltpu.sync_copy(src_all.at[c, s], src_v)
        pltpu.sync_copy(dst_all.at[c, s], dst_v)

        def stripe_out(src_ref, dst_ref):
            obase = s * _OST
            pltpu.sync_copy(src_ref.at[pl.ds(obase, _OST)],
                            dst_ref.at[c, pl.ds(obase, _OST)])

            @pl.when(s == _NT - 1)
            def _():
                tbase = _NT * _OST
                pltpu.sync_copy(src_ref.at[pl.ds(tbase, _N - _NT * _OST)],
                                dst_ref.at[c, pl.ds(tbase, _N - _NT * _OST)])

        def half_pass(tab, out, count_pass):
            if count_pass:
                _fill_f32(ones_v, _CK, _CW, 0.0)
            zbase = s * _ZST
            for b in range(_ZST // _CK):
                pltpu.sync_copy(zbuf, acc.at[pl.ds(zbase + b * _CK, _CK)])
                if count_pass:
                    pltpu.sync_copy(ones_v,
                                    cacc.at[pl.ds(zbase + b * _CK, _CK)])
            if count_pass:
                _fill_f32(ones_v, _CK, _CW, 1.0)
            plsc.subcore_barrier()

            if with_counts:
                rows0, rows1 = rows[0], rows[1]
                sem0, sem1 = gsem[0], gsem[1]
                pltpu.async_copy(tab.at[src_v.at[0]], rows0, sem0)

                def pair(jj, carry):
                    j0 = 2 * jj
                    pltpu.async_copy(tab.at[src_v.at[j0 + 1]], rows1, sem1)
                    pltpu.make_async_copy(tab.at[src_v.at[j0]], rows0,
                                          sem0).wait()
                    pltpu.sync_copy(rows0, acc.at[dst_v.at[j0]], add=True)
                    if count_pass:
                        pltpu.sync_copy(ones_v, cacc.at[dst_v.at[j0]],
                                        add=True)
                    jn = jnp.minimum(j0 + 2, _NCH - 1)
                    pltpu.async_copy(tab.at[src_v.at[jn]], rows0, sem0)
                    pltpu.make_async_copy(tab.at[src_v.at[j0 + 1]], rows1,
                                          sem1).wait()
                    pltpu.sync_copy(rows1, acc.at[dst_v.at[j0 + 1]], add=True)
                    if count_pass:
                        pltpu.sync_copy(ones_v, cacc.at[dst_v.at[j0 + 1]],
                                        add=True)
                    return carry

                lax.fori_loop(0, _NCH // 2, pair, 0)
                pltpu.make_async_copy(tab.at[src_v.at[_NCH - 1]], rows0,
                                      sem0).wait()
                plsc.subcore_barrier()
                stripe_out(acc, out)
                if count_pass:
                    stripe_out(cacc, out_cnt)
                plsc.subcore_barrier()
                return

            def wait_gather(j, b):
                pltpu.make_async_copy(tab.at[src_v.at[j]], rows[b],
                                      gsem[b]).wait()

            def wait_scatter(j, b):
                pltpu.make_async_copy(rows[b], acc.at[dst_v.at[j]],
                                      ssem[b]).wait()

            def step(j, b):
                wait_gather(j, b)
                pltpu.async_copy(rows[b], acc.at[dst_v.at[j]], ssem[b],
                                 add=True)
                b2 = (b + 2) % 4
                wait_scatter(jnp.maximum(j - 2, 0), b2)
                jn = jnp.minimum(j + 2, _NCH - 1)
                pltpu.async_copy(tab.at[src_v.at[jn]], rows[b2], gsem[b2])

            pltpu.async_copy(tab.at[src_v.at[0]], rows[0], gsem[0])
            pltpu.async_copy(tab.at[src_v.at[1]], rows[1], gsem[1])
            pltpu.async_copy(zbuf, acc.at[dst_v.at[0]], ssem[2], add=True)
            pltpu.async_copy(zbuf, acc.at[dst_v.at[0]], ssem[3], add=True)

            def quad(q, carry):
                j0 = 4 * q
                for b in range(4):
                    step(j0 + b, b)
                return carry

            lax.fori_loop(0, _NCH // 4, quad, 0)
            wait_gather(_NCH - 1, 0)
            wait_gather(_NCH - 1, 1)
            wait_scatter(_NCH - 2, 2)
            wait_scatter(_NCH - 1, 3)
            plsc.subcore_barrier()

            stripe_out(acc, out)
            if count_pass:
                stripe_out(cacc, out_cnt)
            plsc.subcore_barrier()

        def both_halves(tab0, tab1):
            half_pass(tab0, out0, with_counts)
            half_pass(tab1, out1, False)

        @pl.when(c == 0)
        def _():
            both_halves(tab_u0, tab_u1)

        @pl.when(c == 1)
        def _():
            both_halves(tab_r0, tab_r1)

    return pl.kernel(body, out_type=out_type, mesh=mesh,
                     scratch_types=scratch,
                     compiler_params=pltpu.CompilerParams(
                         use_tc_tiling_on_sc=False))


_agg_with_counts = _make_agg(True)
_agg_no_counts = _make_agg(False)


_BR = 1000


def _encode_body(xu, xr, Wu, Wr, bu, br, eu, er, ou, outr):
    hp = jax.lax.Precision.HIGHEST
    ou[...] = (jnp.dot(xu[...], Wu[...], precision=hp,
                       preferred_element_type=jnp.float32)
               + bu[...] + eu[...])
    outr[...] = (jnp.dot(xr[...], Wr[...], precision=hp,
                         preferred_element_type=jnp.float32)
                 + br[...] + er[...])


def _encode(xu, xr, Wu, Wr, bu, br, eu, er):
    grid = (_N // _BR,)
    fu = xu.shape[1]
    fr = xr.shape[1]
    return pl.pallas_call(
        _encode_body,
        grid=grid,
        in_specs=[
            pl.BlockSpec((_BR, fu), lambda i: (i, 0)),
            pl.BlockSpec((_BR, fr), lambda i: (i, 0)),
            pl.BlockSpec((fu, _H), lambda i: (0, 0)),
            pl.BlockSpec((fr, _H), lambda i: (0, 0)),
            pl.BlockSpec((1, _H), lambda i: (0, 0)),
            pl.BlockSpec((1, _H), lambda i: (0, 0)),
            pl.BlockSpec((_BR, _H), lambda i: (i, 0)),
            pl.BlockSpec((_BR, _H), lambda i: (i, 0)),
        ],
        out_specs=[
            pl.BlockSpec((_BR, _H), lambda i: (i, 0)),
            pl.BlockSpec((_BR, _H), lambda i: (i, 0)),
        ],
        out_shape=[
            jax.ShapeDtypeStruct((_N, _H), jnp.float32),
            jax.ShapeDtypeStruct((_N, _H), jnp.float32),
        ],
    )(xu, xr, Wu, Wr, bu, br, eu, er)


def _make_combine(apply_relu):
    def body(sums0, sums1, cnts, pu, pr, Wl_u, Wr_u, Wl_r, Wr_r, bl_u, bl_r,
             ou, outr):
        hp = jax.lax.Precision.HIGHEST
        sum_u = jnp.concatenate([sums0[0], sums1[0]], axis=-1)
        sum_r = jnp.concatenate([sums0[1], sums1[1]], axis=-1)
        mean_u = sum_u / jnp.maximum(cnts[0, :, 0:1], 1.0)
        mean_r = sum_r / jnp.maximum(cnts[1, :, 0:1], 1.0)
        hu = (jnp.dot(mean_u, Wl_u[...], precision=hp,
                      preferred_element_type=jnp.float32) + bl_u[...]
              + jnp.dot(pu[...], Wr_u[...], precision=hp,
                        preferred_element_type=jnp.float32))
        hr = (jnp.dot(mean_r, Wl_r[...], precision=hp,
                      preferred_element_type=jnp.float32) + bl_r[...]
              + jnp.dot(pr[...], Wr_r[...], precision=hp,
                        preferred_element_type=jnp.float32))
        if apply_relu:
            hu = jnp.maximum(hu, 0.0)
            hr = jnp.maximum(hr, 0.0)
        ou[...] = hu
        outr[...] = hr

    def run(sums0, sums1, cnts, pu, pr, Wl_u, Wr_u, Wl_r, Wr_r, bl_u, bl_r):
        grid = (_N // _BR,)
        return pl.pallas_call(
            body,
            grid=grid,
            in_specs=[
                pl.BlockSpec((2, _BR, _HH), lambda i: (0, i, 0)),
                pl.BlockSpec((2, _BR, _HH), lambda i: (0, i, 0)),
                pl.BlockSpec((2, _BR, _CW), lambda i: (0, i, 0)),
                pl.BlockSpec((_BR, _H), lambda i: (i, 0)),
                pl.BlockSpec((_BR, _H), lambda i: (i, 0)),
                pl.BlockSpec((_H, _H), lambda i: (0, 0)),
                pl.BlockSpec((_H, _H), lambda i: (0, 0)),
                pl.BlockSpec((_H, _H), lambda i: (0, 0)),
                pl.BlockSpec((_H, _H), lambda i: (0, 0)),
                pl.BlockSpec((1, _H), lambda i: (0, 0)),
                pl.BlockSpec((1, _H), lambda i: (0, 0)),
            ],
            out_specs=[
                pl.BlockSpec((_BR, _H), lambda i: (i, 0)),
                pl.BlockSpec((_BR, _H), lambda i: (i, 0)),
            ],
            out_shape=[
                jax.ShapeDtypeStruct((_N, _H), jnp.float32),
                jax.ShapeDtypeStruct((_N, _H), jnp.float32),
            ],
        )(sums0, sums1, cnts, pu, pr, Wl_u, Wr_u, Wl_r, Wr_r, bl_u, bl_r)

    return run


_combine_relu = _make_combine(True)
_combine_lin = _make_combine(False)


def kernel(x_user, x_recipe, user_node_id, recipe_node_id,
           edge_index_u2r, edge_index_r2u,
           W_user, b_user, W_recipe, b_recipe, emb_user, emb_recipe,
           Wl1_u2r, Wr1_u2r, Wl1_r2u, Wr1_r2u,
           Wl2_u2r, Wr2_u2r, Wl2_r2u, Wr2_r2u,
           bl1_u2r, bl1_r2u, bl2_u2r, bl2_r2u):
    fu = x_user.shape[1]
    fu_pad = (-fu) % 8
    xu = jnp.pad(x_user, ((0, 0), (0, fu_pad)))
    Wu = jnp.pad(W_user, ((0, fu_pad), (0, 0)))
    eu = jnp.take(emb_user, user_node_id, axis=0)
    er = jnp.take(emb_recipe, recipe_node_id, axis=0)

    h_u, h_r = _encode(xu, x_recipe, Wu, W_recipe,
                       b_user.reshape(1, _H), b_recipe.reshape(1, _H),
                       eu, er)

    src_all = jnp.stack([edge_index_r2u[0], edge_index_u2r[0]]
                        ).astype(jnp.int32).reshape(2, _NT, _NCH, _CK)
    dst_all = jnp.stack([edge_index_r2u[1], edge_index_u2r[1]]
                        ).astype(jnp.int32).reshape(2, _NT, _NCH, _CK)

    s1a, s1b, cnts = _agg_with_counts(h_r[:, :_HH], h_r[:, _HH:],
                                      h_u[:, :_HH], h_u[:, _HH:],
                                      src_all, dst_all)
    h_u1, h_r1 = _combine_relu(s1a, s1b, cnts, h_u, h_r,
                               Wl1_r2u, Wr1_r2u, Wl1_u2r, Wr1_u2r,
                               bl1_r2u.reshape(1, _H),
                               bl1_u2r.reshape(1, _H))

    s2a, s2b = _agg_no_counts(h_r1[:, :_HH], h_r1[:, _HH:],
                              h_u1[:, :_HH], h_u1[:, _HH:],
                              src_all, dst_all)
    h_u2, h_r2 = _combine_lin(s2a, s2b, cnts, h_u1, h_r1,
                              Wl2_r2u, Wr2_r2u, Wl2_u2r, Wr2_u2r,
                              bl2_r2u.reshape(1, _H),
                              bl2_u2r.reshape(1, _H))
    return (h_u2, h_r2)


# speedup vs baseline: 8.2415x; 1.1118x over previous
import functools

import jax
import jax.numpy as jnp
from jax import lax
from jax.experimental import pallas as pl
from jax.experimental.pallas import tpu as pltpu
from jax.experimental.pallas import tpu_sc as plsc

_N = 10000
_E = 320000
_H = 128
_NT = 16
_CK = 125
_NCH = _E // _NT // _CK
_CW = 16
_ZST = 625
_OST = 624
_ACC = _NT * _ZST


def _fill_f32(ref, nrows, ncols, val):
    v = jnp.full((16,), val, jnp.float32)

    def row(r, c):
        for cb in range(ncols // 16):
            ref[r, pl.ds(cb * 16, 16)] = v
        return c

    lax.fori_loop(0, nrows, row, 0)


_HH = _H // 2


def _make_agg(with_counts):
    mesh = plsc.VectorSubcoreMesh(
        core_axis_name="c", subcore_axis_name="s", num_cores=2,
        num_subcores=_NT)

    out_type = [jax.ShapeDtypeStruct((2, _N, _HH), jnp.float32),
                jax.ShapeDtypeStruct((2, _N, _HH), jnp.float32)]
    if with_counts:
        out_type.append(jax.ShapeDtypeStruct((2, _N, _CW), jnp.float32))

    scratch = [
        pltpu.VMEM((_NCH, _CK), jnp.int32),
        pltpu.VMEM((_NCH, _CK), jnp.int32),
        [pltpu.VMEM((_CK, _HH), jnp.float32) for _ in range(4)],
        pltpu.VMEM((_CK, _HH), jnp.float32),
        pltpu.VMEM((_CK, _CW), jnp.float32),
        pltpu.VMEM_SHARED((_ACC, _HH), jnp.float32),
        pltpu.VMEM_SHARED((_ACC, _CW), jnp.float32),
        [pltpu.SemaphoreType.DMA for _ in range(4)],
        [pltpu.SemaphoreType.DMA for _ in range(4)],
    ]

    def body(tab_u0, tab_u1, tab_r0, tab_r1, src_all, dst_all, *rest):
        if with_counts:
            (out0, out1, out_cnt, src_v, dst_v, rows, zbuf, ones_v,
             acc, cacc, gsem, ssem) = rest
        else:
            (out0, out1, src_v, dst_v, rows, zbuf, ones_v, acc, cacc,
             gsem, ssem) = rest
            out_cnt = None
        c = lax.axis_index("c")
        s = lax.axis_index("s")
        _fill_f32(zbuf, _CK, _HH, 0.0)

        pltpu.sync_copy(src_all.at[c, s], src_v)
        pltpu.sync_copy(dst_all.at[c, s], dst_v)

        def stripe_out(src_ref, dst_ref):
            obase = s * _OST
            pltpu.sync_copy(src_ref.at[pl.ds(obase, _OST)],
                            dst_ref.at[c, pl.ds(obase, _OST)])

            @pl.when(s == _NT - 1)
            def _():
                tbase = _NT * _OST
                pltpu.sync_copy(src_ref.at[pl.ds(tbase, _N - _NT * _OST)],
                                dst_ref.at[c, pl.ds(tbase, _N - _NT * _OST)])

        def half_pass(tab, out, count_pass):
            if count_pass:
                _fill_f32(ones_v, _CK, _CW, 0.0)
            zbase = s * _ZST
            for b in range(_ZST // _CK):
                pltpu.sync_copy(zbuf, acc.at[pl.ds(zbase + b * _CK, _CK)])
                if count_pass:
                    pltpu.sync_copy(ones_v,
                                    cacc.at[pl.ds(zbase + b * _CK, _CK)])
            if count_pass:
                _fill_f32(ones_v, _CK, _CW, 1.0)
            plsc.subcore_barrier()

            if with_counts:
                rows0, rows1 = rows[0], rows[1]
                sem0, sem1 = gsem[0], gsem[1]
                pltpu.async_copy(tab.at[src_v.at[0]], rows0, sem0)

                def pair(jj, carry):
                    j0 = 2 * jj
                    pltpu.async_copy(tab.at[src_v.at[j0 + 1]], rows1, sem1)
                    pltpu.make_async_copy(tab.at[src_v.at[j0]], rows0,
                                          sem0).wait()
                    pltpu.sync_copy(rows0, acc.at[dst_v.at[j0]], add=True)
                    if count_pass:
                        pltpu.sync_copy(ones_v, cacc.at[dst_v.at[j0]],
                                        add=True)
                    jn = jnp.minimum(j0 + 2, _NCH - 1)
                    pltpu.async_copy(tab.at[src_v.at[jn]], rows0, sem0)
                    pltpu.make_async_copy(tab.at[src_v.at[j0 + 1]], rows1,
                                          sem1).wait()
                    pltpu.sync_copy(rows1, acc.at[dst_v.at[j0 + 1]], add=True)
                    if count_pass:
                        pltpu.sync_copy(ones_v, cacc.at[dst_v.at[j0 + 1]],
                                        add=True)
                    return carry

                lax.fori_loop(0, _NCH // 2, pair, 0)
                pltpu.make_async_copy(tab.at[src_v.at[_NCH - 1]], rows0,
                                      sem0).wait()
                plsc.subcore_barrier()
                stripe_out(acc, out)
                if count_pass:
                    stripe_out(cacc, out_cnt)
                plsc.subcore_barrier()
                return

            def wait_gather(j, b):
                pltpu.make_async_copy(tab.at[src_v.at[j]], rows[b],
                                      gsem[b]).wait()

            def wait_scatter(j, b):
                pltpu.make_async_copy(rows[b], acc.at[dst_v.at[j]],
                                      ssem[b]).wait()

            def step(j, b):
                wait_gather(j, b)
                pltpu.async_copy(rows[b], acc.at[dst_v.at[j]], ssem[b],
                                 add=True)
                b2 = (b + 2) % 4
                wait_scatter(jnp.maximum(j - 2, 0), b2)
                jn = jnp.minimum(j + 2, _NCH - 1)
                pltpu.async_copy(tab.at[src_v.at[jn]], rows[b2], gsem[b2])

            pltpu.async_copy(tab.at[src_v.at[0]], rows[0], gsem[0])
            pltpu.async_copy(tab.at[src_v.at[1]], rows[1], gsem[1])
            pltpu.async_copy(zbuf, acc.at[dst_v.at[0]], ssem[2], add=True)
            pltpu.async_copy(zbuf, acc.at[dst_v.at[0]], ssem[3], add=True)

            def quad(q, carry):
                j0 = 4 * q
                for b in range(4):
                    step(j0 + b, b)
                return carry

            lax.fori_loop(0, _NCH // 4, quad, 0)
            wait_gather(_NCH - 1, 0)
            wait_gather(_NCH - 1, 1)
            wait_scatter(_NCH - 2, 2)
            wait_scatter(_NCH - 1, 3)
            plsc.subcore_barrier()

            stripe_out(acc, out)
            if count_pass:
                stripe_out(cacc, out_cnt)
            plsc.subcore_barrier()

        def both_halves(tab0, tab1):
            half_pass(tab0, out0, with_counts)
            half_pass(tab1, out1, False)

        @pl.when(c == 0)
        def _():
            both_halves(tab_u0, tab_u1)

        @pl.when(c == 1)
        def _():
            both_halves(tab_r0, tab_r1)

    return pl.kernel(body, out_type=out_type, mesh=mesh,
                     scratch_types=scratch,
                     compiler_params=pltpu.CompilerParams(
                         use_tc_tiling_on_sc=False))


_agg_with_counts = _make_agg(True)
_agg_no_counts = _make_agg(False)


_BR = 1000


def _encode_body(xu, xr, Wu, Wr, bu, br, eu, er, ou0, ou1, or0, or1):
    hu = (jnp.dot(xu[...], Wu[...], preferred_element_type=jnp.float32)
          + bu[...] + eu[...])
    hr = (jnp.dot(xr[...], Wr[...], preferred_element_type=jnp.float32)
          + br[...] + er[...])
    ou0[...] = hu[:, :_HH]
    ou1[...] = hu[:, _HH:]
    or0[...] = hr[:, :_HH]
    or1[...] = hr[:, _HH:]


def _encode(xu, xr, Wu, Wr, bu, br, eu, er):
    grid = (_N // _BR,)
    fu = xu.shape[1]
    fr = xr.shape[1]
    half = pl.BlockSpec((_BR, _HH), lambda i: (i, 0))
    return pl.pallas_call(
        _encode_body,
        grid=grid,
        in_specs=[
            pl.BlockSpec((_BR, fu), lambda i: (i, 0)),
            pl.BlockSpec((_BR, fr), lambda i: (i, 0)),
            pl.BlockSpec((fu, _H), lambda i: (0, 0)),
            pl.BlockSpec((fr, _H), lambda i: (0, 0)),
            pl.BlockSpec((1, _H), lambda i: (0, 0)),
            pl.BlockSpec((1, _H), lambda i: (0, 0)),
            pl.BlockSpec((_BR, _H), lambda i: (i, 0)),
            pl.BlockSpec((_BR, _H), lambda i: (i, 0)),
        ],
        out_specs=[half, half, half, half],
        out_shape=[jax.ShapeDtypeStruct((_N, _HH), jnp.float32)] * 4,
    )(xu, xr, Wu, Wr, bu, br, eu, er)


def _make_combine(apply_relu, half_out):
    def body(sums0, sums1, cnts, pu0, pu1, pr0, pr1,
             Wl_u, Wr_u, Wl_r, Wr_r, bl_u, bl_r, *outs):
        sum_u = jnp.concatenate([sums0[0], sums1[0]], axis=-1)
        sum_r = jnp.concatenate([sums0[1], sums1[1]], axis=-1)
        mean_u = sum_u / jnp.maximum(cnts[0, :, 0:1], 1.0)
        mean_r = sum_r / jnp.maximum(cnts[1, :, 0:1], 1.0)
        pu = jnp.concatenate([pu0[...], pu1[...]], axis=-1)
        pr = jnp.concatenate([pr0[...], pr1[...]], axis=-1)
        hu = (jnp.dot(mean_u, Wl_u[...], preferred_element_type=jnp.float32)
              + bl_u[...]
              + jnp.dot(pu, Wr_u[...], preferred_element_type=jnp.float32))
        hr = (jnp.dot(mean_r, Wl_r[...], preferred_element_type=jnp.float32)
              + bl_r[...]
              + jnp.dot(pr, Wr_r[...], preferred_element_type=jnp.float32))
        if apply_relu:
            hu = jnp.maximum(hu, 0.0)
            hr = jnp.maximum(hr, 0.0)
        if half_out:
            outs[0][...] = hu[:, :_HH]
            outs[1][...] = hu[:, _HH:]
            outs[2][...] = hr[:, :_HH]
            outs[3][...] = hr[:, _HH:]
        else:
            outs[0][...] = hu
            outs[1][...] = hr

    def run(sums0, sums1, cnts, pu0, pu1, pr0, pr1,
            Wl_u, Wr_u, Wl_r, Wr_r, bl_u, bl_r):
        grid = (_N // _BR,)
        half = pl.BlockSpec((_BR, _HH), lambda i: (i, 0))
        full = pl.BlockSpec((_BR, _H), lambda i: (i, 0))
        if half_out:
            out_specs = [half] * 4
            out_shape = [jax.ShapeDtypeStruct((_N, _HH), jnp.float32)] * 4
        else:
            out_specs = [full] * 2
            out_shape = [jax.ShapeDtypeStruct((_N, _H), jnp.float32)] * 2
        return pl.pallas_call(
            body,
            grid=grid,
            in_specs=[
                pl.BlockSpec((2, _BR, _HH), lambda i: (0, i, 0)),
                pl.BlockSpec((2, _BR, _HH), lambda i: (0, i, 0)),
                pl.BlockSpec((2, _BR, _CW), lambda i: (0, i, 0)),
                half, half, half, half,
                pl.BlockSpec((_H, _H), lambda i: (0, 0)),
                pl.BlockSpec((_H, _H), lambda i: (0, 0)),
                pl.BlockSpec((_H, _H), lambda i: (0, 0)),
                pl.BlockSpec((_H, _H), lambda i: (0, 0)),
                pl.BlockSpec((1, _H), lambda i: (0, 0)),
                pl.BlockSpec((1, _H), lambda i: (0, 0)),
            ],
            out_specs=out_specs,
            out_shape=out_shape,
        )(sums0, sums1, cnts, pu0, pu1, pr0, pr1,
          Wl_u, Wr_u, Wl_r, Wr_r, bl_u, bl_r)

    return run


_combine_relu = _make_combine(True, True)
_combine_lin = _make_combine(False, False)


def kernel(x_user, x_recipe, user_node_id, recipe_node_id,
           edge_index_u2r, edge_index_r2u,
           W_user, b_user, W_recipe, b_recipe, emb_user, emb_recipe,
           Wl1_u2r, Wr1_u2r, Wl1_r2u, Wr1_r2u,
           Wl2_u2r, Wr2_u2r, Wl2_r2u, Wr2_r2u,
           bl1_u2r, bl1_r2u, bl2_u2r, bl2_r2u):
    fu = x_user.shape[1]
    fu_pad = (-fu) % 8
    xu = jnp.pad(x_user, ((0, 0), (0, fu_pad)))
    Wu = jnp.pad(W_user, ((0, fu_pad), (0, 0)))
    eu = jnp.take(emb_user, user_node_id, axis=0)
    er = jnp.take(emb_recipe, recipe_node_id, axis=0)

    hu0, hu1, hr0, hr1 = _encode(xu, x_recipe, Wu, W_recipe,
                                 b_user.reshape(1, _H),
                                 b_recipe.reshape(1, _H), eu, er)

    src_all = jnp.stack([edge_index_r2u[0], edge_index_u2r[0]]
                        ).astype(jnp.int32).reshape(2, _NT, _NCH, _CK)
    dst_all = jnp.stack([edge_index_r2u[1], edge_index_u2r[1]]
                        ).astype(jnp.int32).reshape(2, _NT, _NCH, _CK)

    s1a, s1b, cnts = _agg_with_counts(hr0, hr1, hu0, hu1, src_all, dst_all)
    u1a, u1b, r1a, r1b = _combine_relu(s1a, s1b, cnts, hu0, hu1, hr0, hr1,
                                       Wl1_r2u, Wr1_r2u, Wl1_u2r, Wr1_u2r,
                                       bl1_r2u.reshape(1, _H),
                                       bl1_u2r.reshape(1, _H))

    s2a, s2b = _agg_no_counts(r1a, r1b, u1a, u1b, src_all, dst_all)
    h_u2, h_r2 = _combine_lin(s2a, s2b, cnts, u1a, u1b, r1a, r1b,
                              Wl2_r2u, Wr2_r2u, Wl2_u2r, Wr2_u2r,
                              bl2_r2u.reshape(1, _H),
                              bl2_u2r.reshape(1, _H))
    return (h_u2, h_r2)
